# Initial kernel scaffold; baseline (speedup 1.0000x reference)
import jax
import jax.numpy as jnp
from jax.experimental import pallas as pl


def kernel(atom_embs, edge_indices, pos, Wi, bi, Wj, bj, Wk2f, Wd, bd, u, rW1, rb1, rW2, rb2):
    raise NotImplementedError("write your pallas kernel here")


# speedup vs baseline: 4.3708x; 4.3708x over previous
import functools

import numpy as np
import jax
import jax.numpy as jnp
from jax import lax
from jax.experimental import pallas as pl
from jax.experimental.pallas import tpu as pltpu
from jax.experimental.pallas import tpu_sc as plsc

_N = 10000
_E = 320000
_F = 128
_K = 64
_CUTOFF = 10.0
_WIDTH = float((0.5 / ((1.0 - np.exp(-_CUTOFF)) / _K)) ** 2)
_CENTERS = np.linspace(1.0, np.exp(-_CUTOFF), _K).astype(np.float32)

_NC = 2
_NS = 16
_NW = _NC * _NS
_EW = _E // _NW
_BC = 80
_NIT = _EW // _BC
_FP = _F + 16
_RN = _N // _NS

_HIGH = lax.Precision.HIGHEST


def _dot(a, b):
    return jax.lax.dot_general(a, b, (((1,), (0,)), ((), ())),
                               precision=_HIGH,
                               preferred_element_type=jnp.float32)


def _sc_d2_body(src_hbm, dst_hbm, px_hbm, py_hbm, pz_hbm, d2_hbm,
                px_v, py_v, pz_v, src_v, dst_v, d2_v):
    cid = lax.axis_index("c")
    sid = lax.axis_index("s")
    wid = sid * _NC + cid
    base = wid * _EW
    pltpu.sync_copy(px_hbm, px_v)
    pltpu.sync_copy(py_hbm, py_v)
    pltpu.sync_copy(pz_hbm, pz_v)
    pltpu.sync_copy(src_hbm.at[pl.ds(base, _EW)], src_v)
    pltpu.sync_copy(dst_hbm.at[pl.ds(base, _EW)], dst_v)

    @pl.loop(0, _EW // 16)
    def _edge16(i):
        o = i * 16
        s16 = src_v[pl.ds(o, 16)]
        d16 = dst_v[pl.ds(o, 16)]
        dx = plsc.load_gather(px_v, [s16]) - plsc.load_gather(px_v, [d16])
        dy = plsc.load_gather(py_v, [s16]) - plsc.load_gather(py_v, [d16])
        dz = plsc.load_gather(pz_v, [s16]) - plsc.load_gather(pz_v, [d16])
        d2_v[pl.ds(o, 16)] = dx * dx + dy * dy + dz * dz

    pltpu.sync_copy(d2_v, d2_hbm.at[pl.ds(base, _EW)])


def _sc_d2(src, dst, px, py, pz):
    mesh = plsc.VectorSubcoreMesh(core_axis_name="c", subcore_axis_name="s")
    return pl.kernel(
        _sc_d2_body,
        out_type=jax.ShapeDtypeStruct((_E,), jnp.float32),
        mesh=mesh,
        scratch_types=[
            pltpu.VMEM((_N,), jnp.float32),
            pltpu.VMEM((_N,), jnp.float32),
            pltpu.VMEM((_N,), jnp.float32),
            pltpu.VMEM((_EW,), jnp.int32),
            pltpu.VMEM((_EW,), jnp.int32),
            pltpu.VMEM((_EW,), jnp.float32),
        ],
        compiler_params=pltpu.CompilerParams(needs_layout_passes=False),
    )(src, dst, px, py, pz)


def _tc_hj_body(a_ref, w_ref, b_ref, o_ref):
    o_ref[...] = _dot(a_ref[...], w_ref[...]) + b_ref[...]


def _tc_hj(atom_embs, Wj, bj):
    bn = 2000
    return pl.pallas_call(
        _tc_hj_body,
        grid=(_N // bn,),
        in_specs=[
            pl.BlockSpec((bn, _F), lambda i: (i, 0)),
            pl.BlockSpec((_F, _F), lambda i: (0, 0)),
            pl.BlockSpec((1, _F), lambda i: (0, 0)),
        ],
        out_specs=pl.BlockSpec((bn, _F), lambda i: (i, 0)),
        out_shape=jax.ShapeDtypeStruct((_N, _F), jnp.float32),
    )(atom_embs, Wj, bj)


_RB = 20


def _tc_g_body(d2_ref, w_ref, g_ref):
    dt = d2_ref[0]
    eye = (lax.broadcasted_iota(jnp.int32, (128, 128), 0)
           == lax.broadcasted_iota(jnp.int32, (128, 128), 1)
           ).astype(jnp.float32)
    tp = jax.lax.dot_general(eye, dt, (((1,), (1,)), ((), ())),
                             precision=_HIGH,
                             preferred_element_type=jnp.float32)
    w = w_ref[...]
    ks = lax.broadcasted_iota(jnp.int32, (1, _K), 1).astype(jnp.float32)
    centers = 1.0 + ks * ((np.exp(-_CUTOFF) - 1.0) / (_K - 1))
    for s in range(_RB):
        d2c = tp[:, s:s + 1]
        dist = jnp.sqrt(d2c + 1e-12)
        x = dist * (1.0 / _CUTOFF)
        x3 = x * x * x
        x4 = x3 * x
        x5 = x4 * x
        cut = jnp.where(x < 1.0, 1.0 - 6.0 * x5 + 15.0 * x4 - 10.0 * x3, 0.0)
        ed = jnp.exp(-dist)
        t = ed - centers
        rbf = cut * jnp.exp(-_WIDTH * (t * t))
        g_ref[0, pl.ds(s * 128, 128), :] = _dot(rbf, w)


def _tc_g(d2_mat, Wk2f):
    nb = _E // (128 * _RB)
    out = pl.pallas_call(
        _tc_g_body,
        grid=(nb,),
        in_specs=[
            pl.BlockSpec((1, _RB, 128), lambda i: (i, 0, 0)),
            pl.BlockSpec((_K, _F), lambda i: (0, 0)),
        ],
        out_specs=pl.BlockSpec((1, 128 * _RB, _F), lambda i: (i, 0, 0)),
        out_shape=jax.ShapeDtypeStruct((nb, 128 * _RB, _F), jnp.float32),
    )(d2_mat, Wk2f)
    return out.reshape(_E, _F)


def _sc_scatter_body(src_hbm, dst_hbm, g_hbm, hj_hbm, sval_hbm, scnt_hbm,
                     src_v, dst_v, g_v, hj_v, prod_v, zero_v, s_sh, sem):
    cid = lax.axis_index("c")
    sid = lax.axis_index("s")
    wid = sid * _NC + cid

    unit = (lax.iota(jnp.int32, 16) == 0).astype(jnp.float32)
    zeros = jnp.zeros((16,), jnp.float32)
    for r in range(_BC):
        prod_v[r, pl.ds(_F, 16)] = unit
    for r in range(25):
        for c in range(_FP // 16):
            zero_v[r, pl.ds(c * 16, 16)] = zeros

    @pl.loop(0, _RN // 25)
    def _zero(k):
        pltpu.sync_copy(zero_v, s_sh.at[pl.ds(sid * _RN + k * 25, 25), :])

    plsc.subcore_barrier()

    @pl.loop(0, _NIT)
    def _chunk(it):
        base = wid * _EW + it * _BC
        pltpu.sync_copy(src_hbm.at[pl.ds(base, _BC)], src_v)
        pltpu.sync_copy(dst_hbm.at[pl.ds(base, _BC)], dst_v)
        pltpu.sync_copy(g_hbm.at[pl.ds(base, _BC), :], g_v)
        pltpu.async_copy(hj_hbm.at[src_v], hj_v, sem).wait()

        @pl.loop(0, _BC)
        def _row(r):
            for c in range(_F // 16):
                sl = pl.ds(c * 16, 16)
                prod_v[r, sl] = g_v[r, sl] * hj_v[r, sl]

        pltpu.sync_copy(prod_v, s_sh.at[dst_v], add=True)

    plsc.subcore_barrier()
    rows = pl.ds(sid * _RN, _RN)
    pltpu.sync_copy(s_sh.at[rows, pl.ds(0, _F)], sval_hbm.at[cid, rows, :])
    pltpu.sync_copy(s_sh.at[rows, pl.ds(_F, 16)], scnt_hbm.at[cid, rows, :])


def _sc_scatter(src, dst, g, hj):
    mesh = plsc.VectorSubcoreMesh(core_axis_name="c", subcore_axis_name="s")
    return pl.kernel(
        _sc_scatter_body,
        out_type=[
            jax.ShapeDtypeStruct((_NC, _N, _F), jnp.float32),
            jax.ShapeDtypeStruct((_NC, _N, 16), jnp.float32),
        ],
        mesh=mesh,
        scratch_types=[
            pltpu.VMEM((_BC,), jnp.int32),
            pltpu.VMEM((_BC,), jnp.int32),
            pltpu.VMEM((_BC, _F), jnp.float32),
            pltpu.VMEM((_BC, _F), jnp.float32),
            pltpu.VMEM((_BC, _FP), jnp.float32),
            pltpu.VMEM((25, _FP), jnp.float32),
            pltpu.VMEM_SHARED((_N, _FP), jnp.float32),
            pltpu.SemaphoreType.DMA,
        ],
        compiler_params=pltpu.CompilerParams(needs_layout_passes=False,
                                             use_tc_tiling_on_sc=False),
    )(src, dst, g, hj)


def _tc_fin_body(a_ref, s_ref, c_ref, wi_ref, bi_ref, u_ref,
                 w1_ref, w2_ref, wd_ref, dv_ref, o_ref):
    w1 = w1_ref[...]
    w2 = w2_ref[...]
    wd = wd_ref[...]
    eye = (lax.broadcasted_iota(jnp.int32, (_F, _F), 0)
           == lax.broadcasted_iota(jnp.int32, (_F, _F), 1)
           ).astype(jnp.float32)
    m = eye + _dot(w1, w2)
    p = _dot(_dot(m, wd), m)
    a = a_ref[...]
    ri = _dot(a, wi_ref[...]) + bi_ref[...]
    s = s_ref[0] + s_ref[1]
    cnt = jnp.sum(c_ref[0] + c_ref[1], axis=1, keepdims=True)
    base = _dot(u_ref[...] * ri, m) + _dot(ri, p) + dv_ref[...]
    o_ref[...] = cnt * base + _dot(s, p)


def _tc_fin(atom_embs, sval, scnt, Wi, bi, u, rW1, rW2, Wd, dvec):
    bn = 2000
    full = lambda i: (0, 0)
    return pl.pallas_call(
        _tc_fin_body,
        grid=(_N // bn,),
        in_specs=[
            pl.BlockSpec((bn, _F), lambda i: (i, 0)),
            pl.BlockSpec((_NC, bn, _F), lambda i: (0, i, 0)),
            pl.BlockSpec((_NC, bn, 16), lambda i: (0, i, 0)),
            pl.BlockSpec((_F, _F), full),
            pl.BlockSpec((1, _F), full),
            pl.BlockSpec((1, _F), full),
            pl.BlockSpec((_F, _F), full),
            pl.BlockSpec((_F, _F), full),
            pl.BlockSpec((_F, _F), full),
            pl.BlockSpec((1, _F), full),
        ],
        out_specs=pl.BlockSpec((bn, _F), lambda i: (i, 0)),
        out_shape=jax.ShapeDtypeStruct((_N, _F), jnp.float32),
    )(atom_embs, sval, scnt, Wi, bi, u, rW1, rW2, Wd, dvec)


def kernel(atom_embs, edge_indices, pos, Wi, bi, Wj, bj, Wk2f, Wd, bd, u,
           rW1, rb1, rW2, rb2):
    src = edge_indices[0]
    dst = edge_indices[1]
    d2 = _sc_d2(src, dst, pos[:, 0], pos[:, 1], pos[:, 2])
    hj = _tc_hj(atom_embs, Wj, bj.reshape(1, _F))
    g = _tc_g(d2.reshape(_E // (128 * _RB), _RB, 128), Wk2f)
    sval, scnt = _sc_scatter(src, dst, g, hj)
    mmat = jnp.eye(_F, dtype=jnp.float32) + rW1[0] @ rW2[0]
    cvec = rb1[0] @ rW2[0] + rb2[0]
    dvec = ((cvec @ Wd + bd) @ mmat + cvec).reshape(1, _F)
    out = _tc_fin(atom_embs, sval, scnt, Wi, bi.reshape(1, _F),
                  u.reshape(1, _F), rW1[0], rW2[0], Wd, dvec)
    return out


# speedup vs baseline: 5.2119x; 1.1924x over previous
import functools

import numpy as np
import jax
import jax.numpy as jnp
from jax import lax
from jax.experimental import pallas as pl
from jax.experimental.pallas import tpu as pltpu
from jax.experimental.pallas import tpu_sc as plsc

_N = 10000
_E = 320000
_F = 128
_K = 64
_CUTOFF = 10.0
_WIDTH = float((0.5 / ((1.0 - np.exp(-_CUTOFF)) / _K)) ** 2)
_CENTERS = np.linspace(1.0, np.exp(-_CUTOFF), _K).astype(np.float32)

_NC = 2
_NS = 16
_NW = _NC * _NS
_EW = _E // _NW
_BC = 80
_NIT = _EW // _BC
_FP = _F + 16
_RN = _N // _NS

_HIGH = lax.Precision.HIGHEST


def _dot(a, b):
    return jax.lax.dot_general(a, b, (((1,), (0,)), ((), ())),
                               precision=_HIGH,
                               preferred_element_type=jnp.float32)


def _sc_d2_body(src_hbm, dst_hbm, px_hbm, py_hbm, pz_hbm, d2_hbm,
                px_v, py_v, pz_v, src_v, dst_v, d2_v):
    cid = lax.axis_index("c")
    sid = lax.axis_index("s")
    wid = sid * _NC + cid
    base = wid * _EW
    pltpu.sync_copy(px_hbm, px_v)
    pltpu.sync_copy(py_hbm, py_v)
    pltpu.sync_copy(pz_hbm, pz_v)
    pltpu.sync_copy(src_hbm.at[pl.ds(base, _EW)], src_v)
    pltpu.sync_copy(dst_hbm.at[pl.ds(base, _EW)], dst_v)

    @pl.loop(0, _EW // 16)
    def _edge16(i):
        o = i * 16
        s16 = src_v[pl.ds(o, 16)]
        d16 = dst_v[pl.ds(o, 16)]
        dx = plsc.load_gather(px_v, [s16]) - plsc.load_gather(px_v, [d16])
        dy = plsc.load_gather(py_v, [s16]) - plsc.load_gather(py_v, [d16])
        dz = plsc.load_gather(pz_v, [s16]) - plsc.load_gather(pz_v, [d16])
        d2_v[pl.ds(o, 16)] = dx * dx + dy * dy + dz * dz

    pltpu.sync_copy(d2_v, d2_hbm.at[pl.ds(base, _EW)])


def _sc_d2(src, dst, px, py, pz):
    mesh = plsc.VectorSubcoreMesh(core_axis_name="c", subcore_axis_name="s")
    return pl.kernel(
        _sc_d2_body,
        out_type=jax.ShapeDtypeStruct((_E,), jnp.float32),
        mesh=mesh,
        scratch_types=[
            pltpu.VMEM((_N,), jnp.float32),
            pltpu.VMEM((_N,), jnp.float32),
            pltpu.VMEM((_N,), jnp.float32),
            pltpu.VMEM((_EW,), jnp.int32),
            pltpu.VMEM((_EW,), jnp.int32),
            pltpu.VMEM((_EW,), jnp.float32),
        ],
        compiler_params=pltpu.CompilerParams(needs_layout_passes=False),
    )(src, dst, px, py, pz)


def _tc_hj_body(a_ref, w_ref, b_ref, o_ref):
    o_ref[...] = _dot(a_ref[...], w_ref[...]) + b_ref[...]


def _tc_hj(atom_embs, Wj, bj):
    bn = 2000
    return pl.pallas_call(
        _tc_hj_body,
        grid=(_N // bn,),
        in_specs=[
            pl.BlockSpec((bn, _F), lambda i: (i, 0)),
            pl.BlockSpec((_F, _F), lambda i: (0, 0)),
            pl.BlockSpec((1, _F), lambda i: (0, 0)),
        ],
        out_specs=pl.BlockSpec((bn, _F), lambda i: (i, 0)),
        out_shape=jax.ShapeDtypeStruct((_N, _F), jnp.float32),
    )(atom_embs, Wj, bj)


_RB = 20


def _tc_g_body(d2_ref, w_ref, g_ref):
    dt = d2_ref[0]
    eye = (lax.broadcasted_iota(jnp.int32, (128, 128), 0)
           == lax.broadcasted_iota(jnp.int32, (128, 128), 1)
           ).astype(jnp.float32)
    tp = jax.lax.dot_general(eye, dt, (((1,), (1,)), ((), ())),
                             precision=_HIGH,
                             preferred_element_type=jnp.float32)
    w = w_ref[...]
    ks = lax.broadcasted_iota(jnp.int32, (1, _K), 1).astype(jnp.float32)
    centers = 1.0 + ks * ((np.exp(-_CUTOFF) - 1.0) / (_K - 1))
    for s in range(_RB):
        d2c = tp[:, s:s + 1]
        dist = jnp.sqrt(d2c + 1e-12)
        x = dist * (1.0 / _CUTOFF)
        x3 = x * x * x
        x4 = x3 * x
        x5 = x4 * x
        cut = jnp.where(x < 1.0, 1.0 - 6.0 * x5 + 15.0 * x4 - 10.0 * x3, 0.0)
        ed = jnp.exp(-dist)
        t = ed - centers
        rbf = cut * jnp.exp(-_WIDTH * (t * t))
        g_ref[0, pl.ds(s * 128, 128), :] = _dot(rbf, w)


def _tc_g(d2_mat, Wk2f):
    nb = _E // (128 * _RB)
    out = pl.pallas_call(
        _tc_g_body,
        grid=(nb,),
        in_specs=[
            pl.BlockSpec((1, _RB, 128), lambda i: (i, 0, 0)),
            pl.BlockSpec((_K, _F), lambda i: (0, 0)),
        ],
        out_specs=pl.BlockSpec((1, 128 * _RB, _F), lambda i: (i, 0, 0)),
        out_shape=jax.ShapeDtypeStruct((nb, 128 * _RB, _F), jnp.float32),
    )(d2_mat, Wk2f)
    return out.reshape(_E, _F)


def _sc_scatter_body(src_hbm, dst_hbm, g_hbm, hj_hbm, sval_hbm, scnt_hbm,
                     src_v, dst_v, g_v, hj_v, prod_v, zero_v, s_sh,
                     sem, sem2, sem3, sem4):
    cid = lax.axis_index("c")
    sid = lax.axis_index("s")
    wid = sid * _NC + cid

    unit = (lax.iota(jnp.int32, 16) == 0).astype(jnp.float32)
    zeros = jnp.zeros((16,), jnp.float32)
    for r in range(_BC):
        prod_v[r, pl.ds(_F, 16)] = unit
    for r in range(25):
        for c in range(_FP // 16):
            zero_v[r, pl.ds(c * 16, 16)] = zeros

    @pl.loop(0, _RN // 25)
    def _zero(k):
        pltpu.sync_copy(zero_v, s_sh.at[pl.ds(sid * _RN + k * 25, 25), :])

    plsc.subcore_barrier()

    @pl.loop(0, _NIT)
    def _chunk(it):
        base = wid * _EW + it * _BC
        d_s = pltpu.async_copy(src_hbm.at[pl.ds(base, _BC)], src_v, sem)
        d_d = pltpu.async_copy(dst_hbm.at[pl.ds(base, _BC)], dst_v, sem2)
        d_g = pltpu.async_copy(g_hbm.at[pl.ds(base, _BC), :], g_v, sem3)
        d_s.wait()
        d_h = pltpu.async_copy(hj_hbm.at[src_v], hj_v, sem4)
        d_d.wait()
        d_g.wait()
        d_h.wait()

        @pl.loop(0, _BC, unroll=2)
        def _row(r):
            for c in range(_F // 16):
                sl = pl.ds(c * 16, 16)
                prod_v[r, sl] = g_v[r, sl] * hj_v[r, sl]

        pltpu.sync_copy(prod_v, s_sh.at[dst_v], add=True)

    plsc.subcore_barrier()
    rows = pl.ds(sid * _RN, _RN)
    pltpu.sync_copy(s_sh.at[rows, pl.ds(0, _F)], sval_hbm.at[cid, rows, :])
    pltpu.sync_copy(s_sh.at[rows, pl.ds(_F, 16)], scnt_hbm.at[cid, rows, :])


def _sc_scatter(src, dst, g, hj):
    mesh = plsc.VectorSubcoreMesh(core_axis_name="c", subcore_axis_name="s")
    return pl.kernel(
        _sc_scatter_body,
        out_type=[
            jax.ShapeDtypeStruct((_NC, _N, _F), jnp.float32),
            jax.ShapeDtypeStruct((_NC, _N, 16), jnp.float32),
        ],
        mesh=mesh,
        scratch_types=[
            pltpu.VMEM((_BC,), jnp.int32),
            pltpu.VMEM((_BC,), jnp.int32),
            pltpu.VMEM((_BC, _F), jnp.float32),
            pltpu.VMEM((_BC, _F), jnp.float32),
            pltpu.VMEM((_BC, _FP), jnp.float32),
            pltpu.VMEM((25, _FP), jnp.float32),
            pltpu.VMEM_SHARED((_N, _FP), jnp.float32),
            pltpu.SemaphoreType.DMA,
            pltpu.SemaphoreType.DMA,
            pltpu.SemaphoreType.DMA,
            pltpu.SemaphoreType.DMA,
        ],
        compiler_params=pltpu.CompilerParams(needs_layout_passes=False,
                                             use_tc_tiling_on_sc=False),
    )(src, dst, g, hj)


def _tc_fin_body(a_ref, s_ref, c_ref, wi_ref, bi_ref, u_ref,
                 w1_ref, w2_ref, wd_ref, dv_ref, o_ref):
    w1 = w1_ref[...]
    w2 = w2_ref[...]
    wd = wd_ref[...]
    eye = (lax.broadcasted_iota(jnp.int32, (_F, _F), 0)
           == lax.broadcasted_iota(jnp.int32, (_F, _F), 1)
           ).astype(jnp.float32)
    m = eye + _dot(w1, w2)
    p = _dot(_dot(m, wd), m)
    a = a_ref[...]
    ri = _dot(a, wi_ref[...]) + bi_ref[...]
    s = s_ref[0] + s_ref[1]
    cnt = jnp.sum(c_ref[0] + c_ref[1], axis=1, keepdims=True)
    base = _dot(u_ref[...] * ri, m) + _dot(ri, p) + dv_ref[...]
    o_ref[...] = cnt * base + _dot(s, p)


def _tc_fin(atom_embs, sval, scnt, Wi, bi, u, rW1, rW2, Wd, dvec):
    bn = 2000
    full = lambda i: (0, 0)
    return pl.pallas_call(
        _tc_fin_body,
        grid=(_N // bn,),
        in_specs=[
            pl.BlockSpec((bn, _F), lambda i: (i, 0)),
            pl.BlockSpec((_NC, bn, _F), lambda i: (0, i, 0)),
            pl.BlockSpec((_NC, bn, 16), lambda i: (0, i, 0)),
            pl.BlockSpec((_F, _F), full),
            pl.BlockSpec((1, _F), full),
            pl.BlockSpec((1, _F), full),
            pl.BlockSpec((_F, _F), full),
            pl.BlockSpec((_F, _F), full),
            pl.BlockSpec((_F, _F), full),
            pl.BlockSpec((1, _F), full),
        ],
        out_specs=pl.BlockSpec((bn, _F), lambda i: (i, 0)),
        out_shape=jax.ShapeDtypeStruct((_N, _F), jnp.float32),
    )(atom_embs, sval, scnt, Wi, bi, u, rW1, rW2, Wd, dvec)


def kernel(atom_embs, edge_indices, pos, Wi, bi, Wj, bj, Wk2f, Wd, bd, u,
           rW1, rb1, rW2, rb2):
    src = edge_indices[0]
    dst = edge_indices[1]
    d2 = _sc_d2(src, dst, pos[:, 0], pos[:, 1], pos[:, 2])
    hj = _tc_hj(atom_embs, Wj, bj.reshape(1, _F))
    g = _tc_g(d2.reshape(_E // (128 * _RB), _RB, 128), Wk2f)
    sval, scnt = _sc_scatter(src, dst, g, hj)
    mmat = jnp.eye(_F, dtype=jnp.float32) + rW1[0] @ rW2[0]
    cvec = rb1[0] @ rW2[0] + rb2[0]
    dvec = ((cvec @ Wd + bd) @ mmat + cvec).reshape(1, _F)
    out = _tc_fin(atom_embs, sval, scnt, Wi, bi.reshape(1, _F),
                  u.reshape(1, _F), rW1[0], rW2[0], Wd, dvec)
    return out


# speedup vs baseline: 5.4582x; 1.0473x over previous
import functools

import numpy as np
import jax
import jax.numpy as jnp
from jax import lax
from jax.experimental import pallas as pl
from jax.experimental.pallas import tpu as pltpu
from jax.experimental.pallas import tpu_sc as plsc

_N = 10000
_E = 320000
_F = 128
_K = 64
_CUTOFF = 10.0
_WIDTH = float((0.5 / ((1.0 - np.exp(-_CUTOFF)) / _K)) ** 2)
_CENTERS = np.linspace(1.0, np.exp(-_CUTOFF), _K).astype(np.float32)

_NC = 2
_NS = 16
_NW = _NC * _NS
_EW = _E // _NW
_BC = 80
_NIT = _EW // _BC
_FP = _F + 16
_RN = _N // _NS

_HIGH = lax.Precision.HIGHEST


def _dot(a, b):
    return jax.lax.dot_general(a, b, (((1,), (0,)), ((), ())),
                               precision=_HIGH,
                               preferred_element_type=jnp.float32)


def _sc_d2_body(src_hbm, dst_hbm, px_hbm, py_hbm, pz_hbm, d2_hbm,
                px_v, py_v, pz_v, src_v, dst_v, d2_v):
    cid = lax.axis_index("c")
    sid = lax.axis_index("s")
    wid = sid * _NC + cid
    base = wid * _EW
    pltpu.sync_copy(px_hbm, px_v)
    pltpu.sync_copy(py_hbm, py_v)
    pltpu.sync_copy(pz_hbm, pz_v)
    pltpu.sync_copy(src_hbm.at[pl.ds(base, _EW)], src_v)
    pltpu.sync_copy(dst_hbm.at[pl.ds(base, _EW)], dst_v)

    @pl.loop(0, _EW // 16)
    def _edge16(i):
        o = i * 16
        s16 = src_v[pl.ds(o, 16)]
        d16 = dst_v[pl.ds(o, 16)]
        dx = plsc.load_gather(px_v, [s16]) - plsc.load_gather(px_v, [d16])
        dy = plsc.load_gather(py_v, [s16]) - plsc.load_gather(py_v, [d16])
        dz = plsc.load_gather(pz_v, [s16]) - plsc.load_gather(pz_v, [d16])
        d2_v[pl.ds(o, 16)] = dx * dx + dy * dy + dz * dz

    pltpu.sync_copy(d2_v, d2_hbm.at[pl.ds(base, _EW)])


def _sc_d2(src, dst, px, py, pz):
    mesh = plsc.VectorSubcoreMesh(core_axis_name="c", subcore_axis_name="s")
    return pl.kernel(
        _sc_d2_body,
        out_type=jax.ShapeDtypeStruct((_E,), jnp.float32),
        mesh=mesh,
        scratch_types=[
            pltpu.VMEM((_N,), jnp.float32),
            pltpu.VMEM((_N,), jnp.float32),
            pltpu.VMEM((_N,), jnp.float32),
            pltpu.VMEM((_EW,), jnp.int32),
            pltpu.VMEM((_EW,), jnp.int32),
            pltpu.VMEM((_EW,), jnp.float32),
        ],
        compiler_params=pltpu.CompilerParams(needs_layout_passes=False),
    )(src, dst, px, py, pz)


def _tc_hj_body(a_ref, w_ref, b_ref, o_ref):
    o_ref[...] = _dot(a_ref[...], w_ref[...]) + b_ref[...]


def _tc_hj(atom_embs, Wj, bj):
    bn = 2000
    return pl.pallas_call(
        _tc_hj_body,
        grid=(_N // bn,),
        in_specs=[
            pl.BlockSpec((bn, _F), lambda i: (i, 0)),
            pl.BlockSpec((_F, _F), lambda i: (0, 0)),
            pl.BlockSpec((1, _F), lambda i: (0, 0)),
        ],
        out_specs=pl.BlockSpec((bn, _F), lambda i: (i, 0)),
        out_shape=jax.ShapeDtypeStruct((_N, _F), jnp.float32),
    )(atom_embs, Wj, bj)


_RB = 20


def _tc_g_body(d2_ref, w_ref, g_ref):
    dt = d2_ref[0]
    eye = (lax.broadcasted_iota(jnp.int32, (128, 128), 0)
           == lax.broadcasted_iota(jnp.int32, (128, 128), 1)
           ).astype(jnp.float32)
    tp = jax.lax.dot_general(eye, dt, (((1,), (1,)), ((), ())),
                             precision=_HIGH,
                             preferred_element_type=jnp.float32)
    w = w_ref[...]
    ks = lax.broadcasted_iota(jnp.int32, (1, _K), 1).astype(jnp.float32)
    centers = 1.0 + ks * ((np.exp(-_CUTOFF) - 1.0) / (_K - 1))
    for s in range(_RB):
        d2c = tp[:, s:s + 1]
        dist = jnp.sqrt(d2c + 1e-12)
        x = dist * (1.0 / _CUTOFF)
        x3 = x * x * x
        x4 = x3 * x
        x5 = x4 * x
        cut = jnp.where(x < 1.0, 1.0 - 6.0 * x5 + 15.0 * x4 - 10.0 * x3, 0.0)
        ed = jnp.exp(-dist)
        t = ed - centers
        rbf = cut * jnp.exp(-_WIDTH * (t * t))
        g_ref[0, pl.ds(s * 128, 128), :] = jax.lax.dot_general(
            rbf, w, (((1,), (0,)), ((), ())),
            preferred_element_type=jnp.float32)


def _tc_g(d2_mat, Wk2f):
    nb = _E // (128 * _RB)
    out = pl.pallas_call(
        _tc_g_body,
        grid=(nb,),
        in_specs=[
            pl.BlockSpec((1, _RB, 128), lambda i: (i, 0, 0)),
            pl.BlockSpec((_K, _F), lambda i: (0, 0)),
        ],
        out_specs=pl.BlockSpec((1, 128 * _RB, _F), lambda i: (i, 0, 0)),
        out_shape=jax.ShapeDtypeStruct((nb, 128 * _RB, _F), jnp.float32),
    )(d2_mat, Wk2f)
    return out.reshape(_E, _F)


def _sc_scatter_body(src_hbm, dst_hbm, g_hbm, hj_hbm, sval_hbm, scnt_hbm,
                     src_v, dst_v, g_v, hj_v, prod_v, zero_v, s_sh,
                     sem, sem2, sem3, sem4):
    cid = lax.axis_index("c")
    sid = lax.axis_index("s")
    wid = sid * _NC + cid

    unit = (lax.iota(jnp.int32, 16) == 0).astype(jnp.float32)
    zeros = jnp.zeros((16,), jnp.float32)
    for r in range(_BC):
        prod_v[r, pl.ds(_F, 16)] = unit
    for r in range(25):
        for c in range(_FP // 16):
            zero_v[r, pl.ds(c * 16, 16)] = zeros

    @pl.loop(0, _RN // 25)
    def _zero(k):
        pltpu.sync_copy(zero_v, s_sh.at[pl.ds(sid * _RN + k * 25, 25), :])

    plsc.subcore_barrier()

    @pl.loop(0, _NIT)
    def _chunk(it):
        base = wid * _EW + it * _BC
        d_s = pltpu.async_copy(src_hbm.at[pl.ds(base, _BC)], src_v, sem)
        d_d = pltpu.async_copy(dst_hbm.at[pl.ds(base, _BC)], dst_v, sem2)
        d_g = pltpu.async_copy(g_hbm.at[pl.ds(base, _BC), :], g_v, sem3)
        d_s.wait()
        d_h = pltpu.async_copy(hj_hbm.at[src_v], hj_v, sem4)
        d_d.wait()
        d_g.wait()
        d_h.wait()

        @pl.loop(0, _BC, unroll=2)
        def _row(r):
            for c in range(_F // 16):
                sl = pl.ds(c * 16, 16)
                prod_v[r, sl] = g_v[r, sl] * hj_v[r, sl]

        pltpu.sync_copy(prod_v, s_sh.at[dst_v], add=True)

    plsc.subcore_barrier()
    rows = pl.ds(sid * _RN, _RN)
    pltpu.sync_copy(s_sh.at[rows, pl.ds(0, _F)], sval_hbm.at[cid, rows, :])
    pltpu.sync_copy(s_sh.at[rows, pl.ds(_F, 16)], scnt_hbm.at[cid, rows, :])


def _sc_scatter(src, dst, g, hj):
    mesh = plsc.VectorSubcoreMesh(core_axis_name="c", subcore_axis_name="s")
    return pl.kernel(
        _sc_scatter_body,
        out_type=[
            jax.ShapeDtypeStruct((_NC, _N, _F), jnp.float32),
            jax.ShapeDtypeStruct((_NC, _N, 16), jnp.float32),
        ],
        mesh=mesh,
        scratch_types=[
            pltpu.VMEM((_BC,), jnp.int32),
            pltpu.VMEM((_BC,), jnp.int32),
            pltpu.VMEM((_BC, _F), jnp.float32),
            pltpu.VMEM((_BC, _F), jnp.float32),
            pltpu.VMEM((_BC, _FP), jnp.float32),
            pltpu.VMEM((25, _FP), jnp.float32),
            pltpu.VMEM_SHARED((_N, _FP), jnp.float32),
            pltpu.SemaphoreType.DMA,
            pltpu.SemaphoreType.DMA,
            pltpu.SemaphoreType.DMA,
            pltpu.SemaphoreType.DMA,
        ],
        compiler_params=pltpu.CompilerParams(needs_layout_passes=False,
                                             use_tc_tiling_on_sc=False),
    )(src, dst, g, hj)


def _tc_fin_body(a_ref, s_ref, c_ref, wi_ref, bi_ref, u_ref,
                 w1_ref, w2_ref, wd_ref, dv_ref, o_ref):
    w1 = w1_ref[...]
    w2 = w2_ref[...]
    wd = wd_ref[...]
    eye = (lax.broadcasted_iota(jnp.int32, (_F, _F), 0)
           == lax.broadcasted_iota(jnp.int32, (_F, _F), 1)
           ).astype(jnp.float32)
    m = eye + _dot(w1, w2)
    p = _dot(_dot(m, wd), m)
    a = a_ref[...]
    ri = _dot(a, wi_ref[...]) + bi_ref[...]
    s = s_ref[0] + s_ref[1]
    cnt = jnp.sum(c_ref[0] + c_ref[1], axis=1, keepdims=True)
    base = _dot(u_ref[...] * ri, m) + _dot(ri, p) + dv_ref[...]
    o_ref[...] = cnt * base + _dot(s, p)


def _tc_fin(atom_embs, sval, scnt, Wi, bi, u, rW1, rW2, Wd, dvec):
    bn = 2000
    full = lambda i: (0, 0)
    return pl.pallas_call(
        _tc_fin_body,
        grid=(_N // bn,),
        in_specs=[
            pl.BlockSpec((bn, _F), lambda i: (i, 0)),
            pl.BlockSpec((_NC, bn, _F), lambda i: (0, i, 0)),
            pl.BlockSpec((_NC, bn, 16), lambda i: (0, i, 0)),
            pl.BlockSpec((_F, _F), full),
            pl.BlockSpec((1, _F), full),
            pl.BlockSpec((1, _F), full),
            pl.BlockSpec((_F, _F), full),
            pl.BlockSpec((_F, _F), full),
            pl.BlockSpec((_F, _F), full),
            pl.BlockSpec((1, _F), full),
        ],
        out_specs=pl.BlockSpec((bn, _F), lambda i: (i, 0)),
        out_shape=jax.ShapeDtypeStruct((_N, _F), jnp.float32),
    )(atom_embs, sval, scnt, Wi, bi, u, rW1, rW2, Wd, dvec)


def kernel(atom_embs, edge_indices, pos, Wi, bi, Wj, bj, Wk2f, Wd, bd, u,
           rW1, rb1, rW2, rb2):
    src = edge_indices[0]
    dst = edge_indices[1]
    d2 = _sc_d2(src, dst, pos[:, 0], pos[:, 1], pos[:, 2])
    hj = _tc_hj(atom_embs, Wj, bj.reshape(1, _F))
    g = _tc_g(d2.reshape(_E // (128 * _RB), _RB, 128), Wk2f)
    sval, scnt = _sc_scatter(src, dst, g, hj)
    mmat = jnp.eye(_F, dtype=jnp.float32) + rW1[0] @ rW2[0]
    cvec = rb1[0] @ rW2[0] + rb2[0]
    dvec = ((cvec @ Wd + bd) @ mmat + cvec).reshape(1, _F)
    out = _tc_fin(atom_embs, sval, scnt, Wi, bi.reshape(1, _F),
                  u.reshape(1, _F), rW1[0], rW2[0], Wd, dvec)
    return out


# speedup vs baseline: 5.4594x; 1.0002x over previous
import numpy as np
import jax
import jax.numpy as jnp
from jax import lax
from jax.experimental import pallas as pl
from jax.experimental.pallas import tpu as pltpu
from jax.experimental.pallas import tpu_sc as plsc

_N = 10000
_E = 320000
_F = 128
_K = 64
_CUTOFF = 10.0
_WIDTH = float((0.5 / ((1.0 - np.exp(-_CUTOFF)) / _K)) ** 2)

_NC = 2
_NS = 16
_NW = _NC * _NS
_EW = _E // _NW
_BC = 80
_NIT = _EW // _BC
_FP = _F + 16
_RN = _N // _NS

_HIGH = lax.Precision.HIGHEST


def _dot(a, b):
    return jax.lax.dot_general(a, b, (((1,), (0,)), ((), ())),
                               precision=_HIGH,
                               preferred_element_type=jnp.float32)


def _sc_d2_body(src_hbm, dst_hbm, px_hbm, py_hbm, pz_hbm, d2_hbm,
                px_v, py_v, pz_v, src_v, dst_v, d2_v):
    cid = lax.axis_index("c")
    sid = lax.axis_index("s")
    wid = sid * _NC + cid
    base = wid * _EW
    pltpu.sync_copy(px_hbm, px_v)
    pltpu.sync_copy(py_hbm, py_v)
    pltpu.sync_copy(pz_hbm, pz_v)
    pltpu.sync_copy(src_hbm.at[pl.ds(base, _EW)], src_v)
    pltpu.sync_copy(dst_hbm.at[pl.ds(base, _EW)], dst_v)

    @pl.loop(0, _EW // 16)
    def _edge16(i):
        o = i * 16
        s16 = src_v[pl.ds(o, 16)]
        d16 = dst_v[pl.ds(o, 16)]
        dx = plsc.load_gather(px_v, [s16]) - plsc.load_gather(px_v, [d16])
        dy = plsc.load_gather(py_v, [s16]) - plsc.load_gather(py_v, [d16])
        dz = plsc.load_gather(pz_v, [s16]) - plsc.load_gather(pz_v, [d16])
        d2_v[pl.ds(o, 16)] = dx * dx + dy * dy + dz * dz

    pltpu.sync_copy(d2_v, d2_hbm.at[pl.ds(base, _EW)])


def _sc_d2(src, dst, px, py, pz):
    mesh = plsc.VectorSubcoreMesh(core_axis_name="c", subcore_axis_name="s")
    return pl.kernel(
        _sc_d2_body,
        out_type=jax.ShapeDtypeStruct((_E,), jnp.float32),
        mesh=mesh,
        scratch_types=[
            pltpu.VMEM((_N,), jnp.float32),
            pltpu.VMEM((_N,), jnp.float32),
            pltpu.VMEM((_N,), jnp.float32),
            pltpu.VMEM((_EW,), jnp.int32),
            pltpu.VMEM((_EW,), jnp.int32),
            pltpu.VMEM((_EW,), jnp.float32),
        ],
        compiler_params=pltpu.CompilerParams(needs_layout_passes=False),
    )(src, dst, px, py, pz)


def _tc_hj_body(a_ref, w_ref, b_ref, o_ref):
    o_ref[...] = _dot(a_ref[...], w_ref[...]) + b_ref[...]


def _tc_hj(atom_embs, Wj, bj):
    bn = 2000
    return pl.pallas_call(
        _tc_hj_body,
        grid=(_N // bn,),
        in_specs=[
            pl.BlockSpec((bn, _F), lambda i: (i, 0)),
            pl.BlockSpec((_F, _F), lambda i: (0, 0)),
            pl.BlockSpec((1, _F), lambda i: (0, 0)),
        ],
        out_specs=pl.BlockSpec((bn, _F), lambda i: (i, 0)),
        out_shape=jax.ShapeDtypeStruct((_N, _F), jnp.float32),
    )(atom_embs, Wj, bj)


_RB = 20


def _tc_g_body(d2_ref, w_ref, g_ref):
    dt = d2_ref[0]
    eye = (lax.broadcasted_iota(jnp.int32, (128, 128), 0)
           == lax.broadcasted_iota(jnp.int32, (128, 128), 1)
           ).astype(jnp.float32)
    tp = jax.lax.dot_general(eye, dt, (((1,), (1,)), ((), ())),
                             precision=_HIGH,
                             preferred_element_type=jnp.float32)
    w = w_ref[...]
    ks = lax.broadcasted_iota(jnp.int32, (1, _K), 1).astype(jnp.float32)
    centers = 1.0 + ks * ((np.exp(-_CUTOFF) - 1.0) / (_K - 1))
    for s in range(_RB):
        d2c = tp[:, s:s + 1]
        dist = jnp.sqrt(d2c + 1e-12)
        x = dist * (1.0 / _CUTOFF)
        x3 = x * x * x
        x4 = x3 * x
        x5 = x4 * x
        cut = jnp.where(x < 1.0, 1.0 - 6.0 * x5 + 15.0 * x4 - 10.0 * x3, 0.0)
        ed = jnp.exp(-dist)
        t = ed - centers
        rbf = cut * jnp.exp(-_WIDTH * (t * t))
        g_ref[0, pl.ds(s * 128, 128), :] = jax.lax.dot_general(
            rbf, w, (((1,), (0,)), ((), ())),
            preferred_element_type=jnp.float32)


def _tc_g(d2_mat, Wk2f):
    nb = _E // (128 * _RB)
    out = pl.pallas_call(
        _tc_g_body,
        grid=(nb,),
        in_specs=[
            pl.BlockSpec((1, _RB, 128), lambda i: (i, 0, 0)),
            pl.BlockSpec((_K, _F), lambda i: (0, 0)),
        ],
        out_specs=pl.BlockSpec((1, 128 * _RB, _F), lambda i: (i, 0, 0)),
        out_shape=jax.ShapeDtypeStruct((nb, 128 * _RB, _F), jnp.float32),
    )(d2_mat, Wk2f)
    return out.reshape(_E, _F)


def _sc_scatter_body(src_hbm, dst_hbm, g_hbm, hj_hbm, sval_hbm, scnt_hbm,
                     src_v, dst_v, g_v, hj_v, prod_v, zero_v, s_sh,
                     sem, sem2, sem3, sem4):
    cid = lax.axis_index("c")
    sid = lax.axis_index("s")
    wid = sid * _NC + cid

    unit = (lax.iota(jnp.int32, 16) == 0).astype(jnp.float32)
    zeros = jnp.zeros((16,), jnp.float32)
    for r in range(_BC):
        prod_v[r, pl.ds(_F, 16)] = unit
    for r in range(25):
        for c in range(_FP // 16):
            zero_v[r, pl.ds(c * 16, 16)] = zeros

    @pl.loop(0, _RN // 25)
    def _zero(k):
        pltpu.sync_copy(zero_v, s_sh.at[pl.ds(sid * _RN + k * 25, 25), :])

    plsc.subcore_barrier()

    @pl.loop(0, _NIT)
    def _chunk(it):
        base = wid * _EW + it * _BC
        d_s = pltpu.async_copy(src_hbm.at[pl.ds(base, _BC)], src_v, sem)
        d_d = pltpu.async_copy(dst_hbm.at[pl.ds(base, _BC)], dst_v, sem2)
        d_g = pltpu.async_copy(g_hbm.at[pl.ds(base, _BC), :], g_v, sem3)
        d_s.wait()
        d_h = pltpu.async_copy(hj_hbm.at[src_v], hj_v, sem4)
        d_d.wait()
        d_g.wait()
        d_h.wait()

        @pl.loop(0, _BC, unroll=2)
        def _row(r):
            for c in range(_F // 16):
                sl = pl.ds(c * 16, 16)
                prod_v[r, sl] = g_v[r, sl] * hj_v[r, sl]

        pltpu.sync_copy(prod_v, s_sh.at[dst_v], add=True)

    plsc.subcore_barrier()
    rows = pl.ds(sid * _RN, _RN)
    pltpu.sync_copy(s_sh.at[rows, pl.ds(0, _F)], sval_hbm.at[cid, rows, :])
    pltpu.sync_copy(s_sh.at[rows, pl.ds(_F, 16)], scnt_hbm.at[cid, rows, :])


def _sc_scatter(src, dst, g, hj):
    mesh = plsc.VectorSubcoreMesh(core_axis_name="c", subcore_axis_name="s")
    return pl.kernel(
        _sc_scatter_body,
        out_type=[
            jax.ShapeDtypeStruct((_NC, _N, _F), jnp.float32),
            jax.ShapeDtypeStruct((_NC, _N, 16), jnp.float32),
        ],
        mesh=mesh,
        scratch_types=[
            pltpu.VMEM((_BC,), jnp.int32),
            pltpu.VMEM((_BC,), jnp.int32),
            pltpu.VMEM((_BC, _F), jnp.float32),
            pltpu.VMEM((_BC, _F), jnp.float32),
            pltpu.VMEM((_BC, _FP), jnp.float32),
            pltpu.VMEM((25, _FP), jnp.float32),
            pltpu.VMEM_SHARED((_N, _FP), jnp.float32),
            pltpu.SemaphoreType.DMA,
            pltpu.SemaphoreType.DMA,
            pltpu.SemaphoreType.DMA,
            pltpu.SemaphoreType.DMA,
        ],
        compiler_params=pltpu.CompilerParams(needs_layout_passes=False,
                                             use_tc_tiling_on_sc=False),
    )(src, dst, g, hj)


def _tc_fin_body(a_ref, s_ref, c_ref, wi_ref, bi_ref, u_ref,
                 w1_ref, w2_ref, wd_ref, dv_ref, o_ref):
    w1 = w1_ref[...]
    w2 = w2_ref[...]
    wd = wd_ref[...]
    eye = (lax.broadcasted_iota(jnp.int32, (_F, _F), 0)
           == lax.broadcasted_iota(jnp.int32, (_F, _F), 1)
           ).astype(jnp.float32)
    m = eye + _dot(w1, w2)
    p = _dot(_dot(m, wd), m)
    a = a_ref[...]
    ri = _dot(a, wi_ref[...]) + bi_ref[...]
    s = s_ref[0] + s_ref[1]
    cnt = jnp.sum(c_ref[0] + c_ref[1], axis=1, keepdims=True)
    base = _dot(u_ref[...] * ri, m) + _dot(ri, p) + dv_ref[...]
    o_ref[...] = cnt * base + _dot(s, p)


def _tc_fin(atom_embs, sval, scnt, Wi, bi, u, rW1, rW2, Wd, dvec):
    bn = 2000
    full = lambda i: (0, 0)
    return pl.pallas_call(
        _tc_fin_body,
        grid=(_N // bn,),
        in_specs=[
            pl.BlockSpec((bn, _F), lambda i: (i, 0)),
            pl.BlockSpec((_NC, bn, _F), lambda i: (0, i, 0)),
            pl.BlockSpec((_NC, bn, 16), lambda i: (0, i, 0)),
            pl.BlockSpec((_F, _F), full),
            pl.BlockSpec((1, _F), full),
            pl.BlockSpec((1, _F), full),
            pl.BlockSpec((_F, _F), full),
            pl.BlockSpec((_F, _F), full),
            pl.BlockSpec((_F, _F), full),
            pl.BlockSpec((1, _F), full),
        ],
        out_specs=pl.BlockSpec((bn, _F), lambda i: (i, 0)),
        out_shape=jax.ShapeDtypeStruct((_N, _F), jnp.float32),
    )(atom_embs, sval, scnt, Wi, bi, u, rW1, rW2, Wd, dvec)


def kernel(atom_embs, edge_indices, pos, Wi, bi, Wj, bj, Wk2f, Wd, bd, u,
           rW1, rb1, rW2, rb2):
    src = edge_indices[0]
    dst = edge_indices[1]
    d2 = _sc_d2(src, dst, pos[:, 0], pos[:, 1], pos[:, 2])
    hj = _tc_hj(atom_embs, Wj, bj.reshape(1, _F))
    g = _tc_g(d2.reshape(_E // (128 * _RB), _RB, 128), Wk2f)
    sval, scnt = _sc_scatter(src, dst, g, hj)
    mmat = jnp.eye(_F, dtype=jnp.float32) + rW1[0] @ rW2[0]
    cvec = rb1[0] @ rW2[0] + rb2[0]
    dvec = ((cvec @ Wd + bd) @ mmat + cvec).reshape(1, _F)
    out = _tc_fin(atom_embs, sval, scnt, Wi, bi.reshape(1, _F),
                  u.reshape(1, _F), rW1[0], rW2[0], Wd, dvec)
    return out
